# baseline (speedup 1.0000x reference)
import functools

import jax
import jax.numpy as jnp
from jax import lax
from jax.experimental import pallas as pl
from jax.experimental.pallas import tpu as pltpu
from jax.experimental.pallas import tpu_sc as plsc

N_NODES = 10000
HID = 128
N_EDGES = 320000

NC = 2
NS = 16
NW = NC * NS

CHUNK = 128
CHUNKS_PER_TILE = 160
EDGES_PER_TILE = CHUNK * CHUNKS_PER_TILE
E_PAD = EDGES_PER_TILE * NS
N_PAD = 10240
ROWS_PER_TILE = N_PAD // NS
HALF = HID // NC

BLK = 1000
NBUF = 5


def _sc_scatter(src_p, dst_p, w_split, zeros_blk):
  mesh = plsc.VectorSubcoreMesh(core_axis_name="c", subcore_axis_name="s")

  @functools.partial(
      pl.kernel,
      out_type=jax.ShapeDtypeStruct((NC, N_PAD, HALF), jnp.float32),
      mesh=mesh,
      scratch_types=[
          pltpu.VMEM((CHUNKS_PER_TILE, CHUNK), jnp.int32),
          pltpu.VMEM((CHUNKS_PER_TILE, CHUNK), jnp.int32),
          [pltpu.VMEM((CHUNK, HALF), jnp.float32)] * NBUF,
          [pltpu.SemaphoreType.DMA] * NBUF,
          [pltpu.SemaphoreType.DMA] * NBUF,
          pltpu.VMEM_SHARED((N_PAD, HALF), jnp.float32),
      ],
      compiler_params=pltpu.CompilerParams(use_tc_tiling_on_sc=False),
  )
  def k(src_hbm, dst_hbm, w_hbm, z_hbm, out_hbm, src_v, dst_v, bufs, gsems,
        ssems, acc):
    c = lax.axis_index("c")
    s = lax.axis_index("s")
    wh = w_hbm.at[c]

    pltpu.sync_copy(z_hbm, bufs[0])
    for kk in range(ROWS_PER_TILE // CHUNK):
      pltpu.sync_copy(bufs[0], acc.at[pl.ds(s * ROWS_PER_TILE + kk * CHUNK,
                                            CHUNK)])
    base = s * CHUNKS_PER_TILE
    pltpu.sync_copy(src_hbm.at[pl.ds(base, CHUNKS_PER_TILE)], src_v)
    pltpu.sync_copy(dst_hbm.at[pl.ds(base, CHUNKS_PER_TILE)], dst_v)
    plsc.subcore_barrier()

    for t in range(NBUF):
      pltpu.async_copy(wh.at[src_v.at[t]], bufs[t], gsems[t])

    def body(g, carry):
      for t in range(NBUF):
        j = g * NBUF + t
        tp = (t - 1) % NBUF
        pltpu.make_async_copy(wh.at[src_v.at[j]], bufs[t], gsems[t]).wait()
        pltpu.async_copy(bufs[t], acc.at[dst_v.at[j]], ssems[t], add=True)

        def drain_refill():
          pltpu.make_async_copy(bufs[tp], acc.at[dst_v.at[j]],
                                ssems[tp]).wait()

          @pl.when(j - 1 + NBUF < CHUNKS_PER_TILE)
          def _():
            pltpu.async_copy(wh.at[src_v.at[j - 1 + NBUF]], bufs[tp],
                             gsems[tp])

        if t == 0:
          pl.when(g >= 1)(drain_refill)
        else:
          drain_refill()
      return carry

    lax.fori_loop(0, CHUNKS_PER_TILE // NBUF, body, 0)
    pltpu.make_async_copy(bufs[(CHUNKS_PER_TILE - 1) % NBUF],
                          acc.at[dst_v.at[0]],
                          ssems[(CHUNKS_PER_TILE - 1) % NBUF]).wait()
    plsc.subcore_barrier()

    for kk in range(ROWS_PER_TILE // CHUNK):
      off = s * ROWS_PER_TILE + kk * CHUNK
      pltpu.sync_copy(acc.at[pl.ds(off, CHUNK)], bufs[kk % NBUF])
      pltpu.sync_copy(bufs[kk % NBUF], out_hbm.at[c, pl.ds(off, CHUNK)])

  return k(src_p, dst_p, w_split, zeros_blk)


def _tc_main_body(acc_ref, x_ref, b_edge_ref, node_W_ref, node_b_ref,
                  cat1_W_ref, cat1_b_ref, cat2_W_ref, cat2_b_ref,
                  f1_W_ref, f1_b_ref, gamma_ref, beta_ref, f2_W_ref, f2_b_ref,
                  out_ref, h_all, stats_ref):
  f32 = jnp.float32
  ph = pl.program_id(0)
  i = pl.program_id(1)

  @pl.when(ph == 0)
  def _():
    A = jnp.concatenate([acc_ref[0], acc_ref[1]], axis=1) + b_edge_ref[...]
    out1 = A + jnp.dot(A, cat1_W_ref[...], preferred_element_type=f32) \
        + cat1_b_ref[...]
    xn = jnp.dot(x_ref[...], node_W_ref[...], preferred_element_type=f32) \
        + node_b_ref[...]
    out2 = out1 + xn \
        + jnp.dot(xn, cat2_W_ref[...], preferred_element_type=f32) \
        + cat2_b_ref[...]
    h = jnp.maximum(out2, 0.0)
    h1 = jnp.dot(h, f1_W_ref[...], preferred_element_type=f32) + f1_b_ref[...]
    h2 = jnp.maximum(h1, 0.0)
    h_all[pl.ds(i * BLK, BLK), :] = h2

    @pl.when(i == 0)
    def _():
      stats_ref[...] = jnp.zeros((8, HID), f32)

    stats_ref[0:1, :] += jnp.sum(h2, axis=0, keepdims=True)
    stats_ref[1:2, :] += jnp.sum(h2 * h2, axis=0, keepdims=True)

  @pl.when(ph == 1)
  def _():
    inv_n = 1.0 / N_NODES
    mean = stats_ref[0:1, :] * inv_n
    var = stats_ref[1:2, :] * inv_n - mean * mean
    scale = gamma_ref[...] * lax.rsqrt(var + 1e-5)
    shift = beta_ref[...] - mean * scale
    hn = h_all[pl.ds(i * BLK, BLK), :] * scale + shift
    out_ref[...] = jnp.dot(hn, f2_W_ref[...],
                           preferred_element_type=f32) + f2_b_ref[...]


def kernel(x, edge_index, W_edge, b_edge, node_W, node_b, cat1_W, cat1_b,
           cat2_W, cat2_b, f1_W, f1_b, bn_gamma, bn_beta, f2_W, f2_b):
  f32 = jnp.float32
  n_extra = E_PAD - N_EDGES
  src_p = jnp.concatenate(
      [edge_index[0], jnp.zeros((n_extra,), jnp.int32)]).reshape(
          NS * CHUNKS_PER_TILE, CHUNK)
  dst_p = jnp.concatenate(
      [edge_index[1], jnp.full((n_extra,), N_NODES, jnp.int32)]).reshape(
          NS * CHUNKS_PER_TILE, CHUNK)
  zeros_blk = jnp.zeros((CHUNK, HALF), f32)
  w_split = jnp.stack([W_edge[:, :HALF], W_edge[:, HALF:]])

  acc = _sc_scatter(src_p, dst_p, w_split, zeros_blk)

  n_blocks = N_NODES // BLK
  row_spec = pl.BlockSpec((BLK, HID), lambda p, i: (i, 0))
  vec_spec = pl.BlockSpec((1, HID), lambda p, i: (0, 0))
  mat_spec = pl.BlockSpec((HID, HID), lambda p, i: (0, 0))

  out = pl.pallas_call(
      _tc_main_body,
      grid=(2, n_blocks),
      in_specs=[
          pl.BlockSpec((NC, BLK, HALF), lambda p, i: (0, i, 0)),
          row_spec,
          vec_spec,
          mat_spec, vec_spec,
          mat_spec, vec_spec,
          mat_spec, vec_spec,
          mat_spec, vec_spec,
          vec_spec, vec_spec,
          mat_spec, vec_spec,
      ],
      out_specs=row_spec,
      out_shape=jax.ShapeDtypeStruct((N_NODES, HID), f32),
      scratch_shapes=[
          pltpu.VMEM((N_NODES, HID), f32),
          pltpu.VMEM((8, HID), f32),
      ],
  )(acc, x, b_edge.reshape(1, HID), node_W, node_b.reshape(1, HID),
    cat1_W, cat1_b.reshape(1, HID), cat2_W, cat2_b.reshape(1, HID),
    f1_W, f1_b.reshape(1, HID), bn_gamma.reshape(1, HID),
    bn_beta.reshape(1, HID), f2_W, f2_b.reshape(1, HID))
  return out


# speedup vs baseline: 1.6288x; 1.6288x over previous
import functools

import jax
import jax.numpy as jnp
from jax import lax
from jax.experimental import pallas as pl
from jax.experimental.pallas import tpu as pltpu
from jax.experimental.pallas import tpu_sc as plsc

N_NODES = 10000
HID = 128
N_EDGES = 320000

NC = 2
NS = 16
NW = NC * NS

CHUNK = 128
CHUNKS_PER_TILE = 160
EDGES_PER_TILE = CHUNK * CHUNKS_PER_TILE
E_PAD = EDGES_PER_TILE * NS
N_PAD = 10240
ROWS_PER_TILE = N_PAD // NS
HALF = HID // NC

BLK = 1000
NBUF = 8


def _sc_scatter(src_p, dst_p, w_split, zeros_blk):
  mesh = plsc.VectorSubcoreMesh(core_axis_name="c", subcore_axis_name="s")

  @functools.partial(
      pl.kernel,
      out_type=jax.ShapeDtypeStruct((NC, N_PAD, HALF), jnp.bfloat16),
      mesh=mesh,
      scratch_types=[
          pltpu.VMEM((CHUNKS_PER_TILE, CHUNK), jnp.int32),
          pltpu.VMEM((CHUNKS_PER_TILE, CHUNK), jnp.int32),
          [pltpu.VMEM((CHUNK, HALF), jnp.bfloat16)] * NBUF,
          [pltpu.SemaphoreType.DMA] * NBUF,
          [pltpu.SemaphoreType.DMA] * NBUF,
          pltpu.VMEM_SHARED((N_PAD, HALF), jnp.bfloat16),
      ],
      compiler_params=pltpu.CompilerParams(use_tc_tiling_on_sc=False),
  )
  def k(src_hbm, dst_hbm, w_hbm, z_hbm, out_hbm, src_v, dst_v, bufs, gsems,
        ssems, acc):
    c = lax.axis_index("c")
    s = lax.axis_index("s")
    wh = w_hbm.at[c]

    pltpu.sync_copy(z_hbm, bufs[0])
    for kk in range(ROWS_PER_TILE // CHUNK):
      pltpu.sync_copy(bufs[0], acc.at[pl.ds(s * ROWS_PER_TILE + kk * CHUNK,
                                            CHUNK)])
    base = s * CHUNKS_PER_TILE
    pltpu.sync_copy(src_hbm.at[pl.ds(base, CHUNKS_PER_TILE)], src_v)
    pltpu.sync_copy(dst_hbm.at[pl.ds(base, CHUNKS_PER_TILE)], dst_v)
    plsc.subcore_barrier()

    for t in range(NBUF):
      pltpu.async_copy(wh.at[src_v.at[t]], bufs[t], gsems[t])

    def body(g, carry):
      for t in range(NBUF):
        j = g * NBUF + t
        tp = (t - 1) % NBUF
        pltpu.make_async_copy(wh.at[src_v.at[j]], bufs[t], gsems[t]).wait()
        pltpu.async_copy(bufs[t], acc.at[dst_v.at[j]], ssems[t], add=True)

        def drain_refill():
          pltpu.make_async_copy(bufs[tp], acc.at[dst_v.at[j]],
                                ssems[tp]).wait()

          @pl.when(j - 1 + NBUF < CHUNKS_PER_TILE)
          def _():
            pltpu.async_copy(wh.at[src_v.at[j - 1 + NBUF]], bufs[tp],
                             gsems[tp])

        if t == 0:
          pl.when(g >= 1)(drain_refill)
        else:
          drain_refill()
      return carry

    lax.fori_loop(0, CHUNKS_PER_TILE // NBUF, body, 0)
    pltpu.make_async_copy(bufs[(CHUNKS_PER_TILE - 1) % NBUF],
                          acc.at[dst_v.at[0]],
                          ssems[(CHUNKS_PER_TILE - 1) % NBUF]).wait()
    plsc.subcore_barrier()

    for kk in range(ROWS_PER_TILE // CHUNK):
      off = s * ROWS_PER_TILE + kk * CHUNK
      pltpu.sync_copy(acc.at[pl.ds(off, CHUNK)], bufs[kk % NBUF])
      pltpu.sync_copy(bufs[kk % NBUF], out_hbm.at[c, pl.ds(off, CHUNK)])

  return k(src_p, dst_p, w_split, zeros_blk)


def _tc_main_body(acc_ref, x_ref, b_edge_ref, node_W_ref, node_b_ref,
                  cat1_W_ref, cat1_b_ref, cat2_W_ref, cat2_b_ref,
                  f1_W_ref, f1_b_ref, gamma_ref, beta_ref, f2_W_ref, f2_b_ref,
                  out_ref, h_all, stats_ref):
  f32 = jnp.float32
  ph = pl.program_id(0)
  i = pl.program_id(1)

  @pl.when(ph == 0)
  def _():
    A = jnp.concatenate([acc_ref[0], acc_ref[1]],
                      axis=1).astype(f32) + b_edge_ref[...]
    out1 = A + jnp.dot(A, cat1_W_ref[...], preferred_element_type=f32) \
        + cat1_b_ref[...]
    xn = jnp.dot(x_ref[...], node_W_ref[...], preferred_element_type=f32) \
        + node_b_ref[...]
    out2 = out1 + xn \
        + jnp.dot(xn, cat2_W_ref[...], preferred_element_type=f32) \
        + cat2_b_ref[...]
    h = jnp.maximum(out2, 0.0)
    h1 = jnp.dot(h, f1_W_ref[...], preferred_element_type=f32) + f1_b_ref[...]
    h2 = jnp.maximum(h1, 0.0)
    h_all[pl.ds(i * BLK, BLK), :] = h2

    @pl.when(i == 0)
    def _():
      stats_ref[...] = jnp.zeros((8, HID), f32)

    stats_ref[0:1, :] += jnp.sum(h2, axis=0, keepdims=True)
    stats_ref[1:2, :] += jnp.sum(h2 * h2, axis=0, keepdims=True)

  @pl.when(ph == 1)
  def _():
    inv_n = 1.0 / N_NODES
    mean = stats_ref[0:1, :] * inv_n
    var = stats_ref[1:2, :] * inv_n - mean * mean
    scale = gamma_ref[...] * lax.rsqrt(var + 1e-5)
    shift = beta_ref[...] - mean * scale
    hn = h_all[pl.ds(i * BLK, BLK), :] * scale + shift
    out_ref[...] = jnp.dot(hn, f2_W_ref[...],
                           preferred_element_type=f32) + f2_b_ref[...]


def kernel(x, edge_index, W_edge, b_edge, node_W, node_b, cat1_W, cat1_b,
           cat2_W, cat2_b, f1_W, f1_b, bn_gamma, bn_beta, f2_W, f2_b):
  f32 = jnp.float32
  n_extra = E_PAD - N_EDGES
  src_p = jnp.concatenate(
      [edge_index[0], jnp.zeros((n_extra,), jnp.int32)]).reshape(
          NS * CHUNKS_PER_TILE, CHUNK)
  dst_p = jnp.concatenate(
      [edge_index[1], jnp.full((n_extra,), N_NODES, jnp.int32)]).reshape(
          NS * CHUNKS_PER_TILE, CHUNK)
  zeros_blk = jnp.zeros((CHUNK, HALF), jnp.bfloat16)
  w_split = jnp.stack(
      [W_edge[:, :HALF], W_edge[:, HALF:]]).astype(jnp.bfloat16)

  acc = _sc_scatter(src_p, dst_p, w_split, zeros_blk)

  n_blocks = N_NODES // BLK
  row_spec = pl.BlockSpec((BLK, HID), lambda p, i: (i, 0))
  vec_spec = pl.BlockSpec((1, HID), lambda p, i: (0, 0))
  mat_spec = pl.BlockSpec((HID, HID), lambda p, i: (0, 0))

  out = pl.pallas_call(
      _tc_main_body,
      grid=(2, n_blocks),
      in_specs=[
          pl.BlockSpec((NC, BLK, HALF), lambda p, i: (0, i, 0)),
          row_spec,
          vec_spec,
          mat_spec, vec_spec,
          mat_spec, vec_spec,
          mat_spec, vec_spec,
          mat_spec, vec_spec,
          vec_spec, vec_spec,
          mat_spec, vec_spec,
      ],
      out_specs=row_spec,
      out_shape=jax.ShapeDtypeStruct((N_NODES, HID), f32),
      scratch_shapes=[
          pltpu.VMEM((N_NODES, HID), f32),
          pltpu.VMEM((8, HID), f32),
      ],
  )(acc, x, b_edge.reshape(1, HID), node_W, node_b.reshape(1, HID),
    cat1_W, cat1_b.reshape(1, HID), cat2_W, cat2_b.reshape(1, HID),
    f1_W, f1_b.reshape(1, HID), bn_gamma.reshape(1, HID),
    bn_beta.reshape(1, HID), f2_W, f2_b.reshape(1, HID))
  return out
